# Initial kernel scaffold; baseline (speedup 1.0000x reference)
import jax
import jax.numpy as jnp
from jax.experimental import pallas as pl


def kernel(protein_pos, protein_v, ligand_pos, ligand_v, batch_protein, batch_ligand, time_step, edge_index, W_p, b_p, W_l, b_l, Wt1, bt1, Wt2, bt2, We1, be1, We2, be2, Wn1, bn1, Wn2, bn2, Wc, bc, Wv, bv):
    raise NotImplementedError("write your pallas kernel here")


# speedup vs baseline: 2.2043x; 2.2043x over previous
import functools

import jax
import jax.numpy as jnp
import numpy as np
from jax.experimental import pallas as pl
from jax.experimental.pallas import tpu as pltpu
from jax.experimental.pallas import tpu_sc as plsc

NPROT, NLIG, NB = 8000, 2000, 16
NND = NPROT + NLIG
EDG, HID, NLAY = 320000, 128, 4
PFD, LFD = 27, 13
BLK_E = 3200

SC_NC, SC_NS, SC_L = 2, 16, 16
NWRK = SC_NC * SC_NS
EPW = EDG // NWRK
CHK = 80
NCHK = EPW // CHK
NPAD = 10240
ROWS_PT = NPAD // SC_NS


def _bdot(a, b):
    return jnp.dot(a.astype(jnp.bfloat16), b.astype(jnp.bfloat16),
                   preferred_element_type=jnp.float32)


def _bf(x):
    return x.astype(jnp.bfloat16).astype(jnp.float32)


_hdot = functools.partial(jnp.dot, preferred_element_type=jnp.float32,
                          precision=jax.lax.Precision.HIGHEST)


def _prologue_body(ppos_ref, pv_ref, lpos_ref, lv_ref, bp_ref, bl_ref, t_ref,
                   freq_ref, Wp_ref, bp_w_ref, Wl_ref, bl_w_ref,
                   Wt1_ref, bt1_ref, Wt2_ref, bt2_ref,
                   h_ref, pos_ref):
    iota_b = jax.lax.broadcasted_iota(jnp.int32, (1, NB), 1)
    oh_p = (bp_ref[...] == iota_b).astype(jnp.float32)
    oh_l = (bl_ref[...] == iota_b).astype(jnp.float32)
    ppos = ppos_ref[...]
    sums = _hdot(oh_p.T, ppos)
    cnts = jnp.sum(oh_p, axis=0, keepdims=True)
    offset = sums / jnp.clip(cnts.T, 1.0, None)
    ppos = ppos - _hdot(oh_p, offset)
    lpos = lpos_ref[...] - _hdot(oh_l, offset)
    t = t_ref[...].astype(jnp.float32)
    arg = t * freq_ref[...]
    emb = jnp.concatenate([jnp.sin(arg), jnp.cos(arg)], axis=1)
    temb = _bdot(jax.nn.relu(_bdot(emb, Wt1_ref[...]) + bt1_ref[...]),
                 Wt2_ref[...]) + bt2_ref[...]
    h_p = _bdot(pv_ref[...], Wp_ref[...]) + bp_w_ref[...]
    h_l = (_bdot(lv_ref[...], Wl_ref[...]) + bl_w_ref[...]
           + _hdot(oh_l, temb))
    h_ref[:NPROT, :] = h_p
    h_ref[NPROT:, :] = h_l
    pos_ref[:NPROT, :] = ppos
    pos_ref[NPROT:, :] = lpos


def _prologue(ppos, pv, lpos, lv, bp, bl, t, freq, Wp, bpw, Wl, blw,
              Wt1, bt1, Wt2, bt2):
    return pl.pallas_call(
        _prologue_body,
        out_shape=[jax.ShapeDtypeStruct((NND, HID), jnp.float32),
                   jax.ShapeDtypeStruct((NND, 16), jnp.float32)],
    )(ppos, pv, lpos, lv, bp, bl, t, freq, Wp, bpw, Wl, blw, Wt1, bt1, Wt2, bt2)


def _prep_body(h_ref, Wa_ref, Wb_ref, be1_ref, ta_ref, tb_ref):
    h = h_ref[...]
    ta_ref[...] = _bdot(h, Wa_ref[...]) + be1_ref[...]
    tb_ref[...] = _bdot(h, Wb_ref[...])


def _prep(h, Wa, Wb, be1):
    return pl.pallas_call(
        _prep_body,
        out_shape=[jax.ShapeDtypeStruct((NND, HID), jnp.float32),
                   jax.ShapeDtypeStruct((NND, HID), jnp.float32)],
    )(h, Wa, Wb, be1)


def _sc_mesh():
    return plsc.VectorSubcoreMesh(core_axis_name="c", subcore_axis_name="s")


def _gather_body(ta_hbm, tb_hbm, px_hbm, py_hbm, pz_hbm, idxd_hbm, idxs_hbm,
                 ga_hbm, gb_hbm, rx_hbm, ry_hbm, rz_hbm, d2_hbm,
                 idxd_v, idxs_v, bufa, bufb, px_v, py_v, pz_v,
                 rx_v, ry_v, rz_v, d2_v, sem):
    cid = jax.lax.axis_index("c")
    sid = jax.lax.axis_index("s")
    wid = cid * SC_NS + sid
    pltpu.sync_copy(idxd_hbm.at[wid], idxd_v)
    pltpu.sync_copy(idxs_hbm.at[wid], idxs_v)
    pltpu.sync_copy(px_hbm, px_v)
    pltpu.sync_copy(py_hbm, py_v)
    pltpu.sync_copy(pz_hbm, pz_v)

    def chunk(c, _):
        base = wid * EPW + c * CHK
        cp1 = pltpu.async_copy(ta_hbm.at[idxd_v.at[c]], bufa, sem)
        cp2 = pltpu.async_copy(tb_hbm.at[idxs_v.at[c]], bufb, sem)

        def jstep(j, _):
            off = c * CHK + j * SC_L
            id_d = idxd_v[c, pl.ds(j * SC_L, SC_L)]
            id_s = idxs_v[c, pl.ds(j * SC_L, SC_L)]
            relx = (plsc.load_gather(px_v, [id_s])
                    - plsc.load_gather(px_v, [id_d]))
            rely = (plsc.load_gather(py_v, [id_s])
                    - plsc.load_gather(py_v, [id_d]))
            relz = (plsc.load_gather(pz_v, [id_s])
                    - plsc.load_gather(pz_v, [id_d]))
            rx_v[pl.ds(off, SC_L)] = relx
            ry_v[pl.ds(off, SC_L)] = rely
            rz_v[pl.ds(off, SC_L)] = relz
            d2_v[pl.ds(off, SC_L)] = relx * relx + rely * rely + relz * relz
            return ()

        jax.lax.fori_loop(0, CHK // SC_L, jstep, ())
        cp1.wait()
        cp2.wait()
        pltpu.sync_copy(bufa, ga_hbm.at[pl.ds(base, CHK)])
        pltpu.sync_copy(bufb, gb_hbm.at[pl.ds(base, CHK)])
        return ()

    jax.lax.fori_loop(0, NCHK, chunk, ())
    base = wid * EPW
    pltpu.sync_copy(rx_v, rx_hbm.at[pl.ds(base, EPW)])
    pltpu.sync_copy(ry_v, ry_hbm.at[pl.ds(base, EPW)])
    pltpu.sync_copy(rz_v, rz_hbm.at[pl.ds(base, EPW)])
    pltpu.sync_copy(d2_v, d2_hbm.at[pl.ds(base, EPW)])


def _gather_stage(ta, tb, px, py, pz, idxd3, idxs3):
    return pl.kernel(
        _gather_body,
        mesh=_sc_mesh(),
        compiler_params=pltpu.CompilerParams(needs_layout_passes=False),
        out_type=[jax.ShapeDtypeStruct((EDG, HID), jnp.float32),
                  jax.ShapeDtypeStruct((EDG, HID), jnp.float32),
                  jax.ShapeDtypeStruct((EDG,), jnp.float32),
                  jax.ShapeDtypeStruct((EDG,), jnp.float32),
                  jax.ShapeDtypeStruct((EDG,), jnp.float32),
                  jax.ShapeDtypeStruct((EDG,), jnp.float32)],
        scratch_types=[pltpu.VMEM((NCHK, CHK), jnp.int32),
                       pltpu.VMEM((NCHK, CHK), jnp.int32),
                       pltpu.VMEM((CHK, HID), jnp.float32),
                       pltpu.VMEM((CHK, HID), jnp.float32),
                       pltpu.VMEM((NND,), jnp.float32),
                       pltpu.VMEM((NND,), jnp.float32),
                       pltpu.VMEM((NND,), jnp.float32),
                       pltpu.VMEM((EPW,), jnp.float32),
                       pltpu.VMEM((EPW,), jnp.float32),
                       pltpu.VMEM((EPW,), jnp.float32),
                       pltpu.VMEM((EPW,), jnp.float32),
                       pltpu.SemaphoreType.DMA],
    )(ta, tb, px, py, pz, idxd3, idxs3)


def _edge_body(ga_ref, gb_ref, rx_ref, ry_ref, rz_ref, d2_ref,
               w2h_ref, We2_ref, be2_ref, wc_ref, bc_ref,
               r_ref, tx_ref, ty_ref, tz_ref):
    d2 = d2_ref[...]
    u = ga_ref[...] + gb_ref[...] + _bf(d2) * _bf(w2h_ref[...])
    r = _bf(jax.nn.relu(u))
    r_ref[...] = r
    ef = _bdot(r, We2_ref[...]) + be2_ref[...]
    dot = (jnp.sum(_bf(ef) * _bf(wc_ref[...]), axis=1, keepdims=True)
           + bc_ref[0, 0])
    coef = jnp.tanh(dot)
    tx_ref[...] = rx_ref[...] * coef
    ty_ref[...] = ry_ref[...] * coef
    tz_ref[...] = rz_ref[...] * coef


def _edge_stage(ga, gb, rx, ry, rz, d2, w2h, We2, be2, wc, bc):
    nblk = EDG // BLK_E
    col1 = pl.BlockSpec((BLK_E, 1), lambda i: (i, 0))
    return pl.pallas_call(
        _edge_body,
        grid=(nblk,),
        in_specs=[
            pl.BlockSpec((BLK_E, HID), lambda i: (i, 0)),
            pl.BlockSpec((BLK_E, HID), lambda i: (i, 0)),
            col1, col1, col1, col1,
            pl.BlockSpec((1, HID), lambda i: (0, 0)),
            pl.BlockSpec((HID, HID), lambda i: (0, 0)),
            pl.BlockSpec((1, HID), lambda i: (0, 0)),
            pl.BlockSpec((1, HID), lambda i: (0, 0)),
            pl.BlockSpec((1, 1), lambda i: (0, 0), memory_space=pltpu.SMEM),
        ],
        out_specs=[pl.BlockSpec((BLK_E, HID), lambda i: (i, 0)),
                   col1, col1, col1],
        out_shape=[jax.ShapeDtypeStruct((EDG, HID), jnp.float32),
                   jax.ShapeDtypeStruct((EDG, 1), jnp.float32),
                   jax.ShapeDtypeStruct((EDG, 1), jnp.float32),
                   jax.ShapeDtypeStruct((EDG, 1), jnp.float32)],
    )(ga, gb, rx, ry, rz, d2, w2h, We2, be2, wc, bc)


def _scatter_r_body(r_hbm, idxd_hbm, ro_hbm,
                    idxd_v, bufr, zr, accr, sem):
    cid = jax.lax.axis_index("c")
    sid = jax.lax.axis_index("s")
    wid = cid * SC_NS + sid
    pltpu.sync_copy(idxd_hbm.at[wid], idxd_v)

    zeros16 = jnp.zeros((SC_L,), jnp.float32)

    def zrow(i, _):
        def zcol(j, _):
            zr[i, pl.ds(j * SC_L, SC_L)] = zeros16
            return ()
        jax.lax.fori_loop(0, HID // SC_L, zcol, ())
        return ()

    jax.lax.fori_loop(0, SC_L, zrow, ())

    def zshared(k, _):
        pltpu.sync_copy(zr, accr.at[pl.ds(sid * ROWS_PT + k * SC_L, SC_L)])
        return ()

    jax.lax.fori_loop(0, ROWS_PT // SC_L, zshared, ())
    plsc.subcore_barrier()

    def chunk(c, _):
        base = wid * EPW + c * CHK
        pltpu.async_copy(r_hbm.at[pl.ds(base, CHK)], bufr, sem).wait()
        pltpu.sync_copy(bufr, accr.at[idxd_v.at[c]], add=True)
        return ()

    jax.lax.fori_loop(0, NCHK, chunk, ())
    plsc.subcore_barrier()

    out0 = sid * ROWS_PT
    pltpu.sync_copy(accr.at[pl.ds(out0, ROWS_PT)],
                    ro_hbm.at[cid, pl.ds(out0, ROWS_PT)])


def _scatter_r_stage(r, idxd3):
    return pl.kernel(
        _scatter_r_body,
        mesh=_sc_mesh(),
        compiler_params=pltpu.CompilerParams(needs_layout_passes=False),
        out_type=jax.ShapeDtypeStruct((SC_NC, NPAD, HID), jnp.float32),
        scratch_types=[pltpu.VMEM((NCHK, CHK), jnp.int32),
                       pltpu.VMEM((CHK, HID), jnp.float32),
                       pltpu.VMEM((SC_L, HID), jnp.float32),
                       pltpu.VMEM_SHARED((NPAD, HID), jnp.float32),
                       pltpu.SemaphoreType.DMA],
    )(r, idxd3)


def _scatter_p_body(tx_hbm, ty_hbm, tz_hbm, idxd_hbm,
                    px_hbm, py_hbm, pz_hbm, pc_hbm,
                    idxd_v, bufx, bufy, bufz,
                    accx, accy, accz, accc, sem):
    cid = jax.lax.axis_index("c")
    sid = jax.lax.axis_index("s")
    wid = cid * SC_NS + sid
    pltpu.sync_copy(idxd_hbm.at[wid], idxd_v)

    zeros16 = jnp.zeros((SC_L,), jnp.float32)

    def zacc(k, _):
        accx[pl.ds(k * SC_L, SC_L)] = zeros16
        accy[pl.ds(k * SC_L, SC_L)] = zeros16
        accz[pl.ds(k * SC_L, SC_L)] = zeros16
        accc[pl.ds(k * SC_L, SC_L)] = zeros16
        return ()

    jax.lax.fori_loop(0, NND // SC_L, zacc, ())

    ones16 = jnp.full((SC_L,), 1.0, jnp.float32)

    def chunk(c, _):
        base = wid * EPW + c * CHK
        cp2 = pltpu.async_copy(tx_hbm.at[pl.ds(base, CHK)], bufx, sem)
        cp3 = pltpu.async_copy(ty_hbm.at[pl.ds(base, CHK)], bufy, sem)
        cp4 = pltpu.async_copy(tz_hbm.at[pl.ds(base, CHK)], bufz, sem)
        cp2.wait()
        cp3.wait()
        cp4.wait()

        def jstep(j, _):
            idv = idxd_v[c, pl.ds(j * SC_L, SC_L)]
            plsc.addupdate_scatter(accx, [idv], bufx[pl.ds(j * SC_L, SC_L)])
            plsc.addupdate_scatter(accy, [idv], bufy[pl.ds(j * SC_L, SC_L)])
            plsc.addupdate_scatter(accz, [idv], bufz[pl.ds(j * SC_L, SC_L)])
            plsc.addupdate_scatter(accc, [idv], ones16)
            return ()

        jax.lax.fori_loop(0, CHK // SC_L, jstep, ())
        return ()

    jax.lax.fori_loop(0, NCHK, chunk, ())

    pltpu.sync_copy(accx, px_hbm.at[wid])
    pltpu.sync_copy(accy, py_hbm.at[wid])
    pltpu.sync_copy(accz, pz_hbm.at[wid])
    pltpu.sync_copy(accc, pc_hbm.at[wid])


def _scatter_p_stage(tx, ty, tz, idxd3):
    return pl.kernel(
        _scatter_p_body,
        mesh=_sc_mesh(),
        compiler_params=pltpu.CompilerParams(needs_layout_passes=False),
        out_type=[jax.ShapeDtypeStruct((NWRK, NND), jnp.float32),
                  jax.ShapeDtypeStruct((NWRK, NND), jnp.float32),
                  jax.ShapeDtypeStruct((NWRK, NND), jnp.float32),
                  jax.ShapeDtypeStruct((NWRK, NND), jnp.float32)],
        scratch_types=[pltpu.VMEM((NCHK, CHK), jnp.int32),
                       pltpu.VMEM((CHK,), jnp.float32),
                       pltpu.VMEM((CHK,), jnp.float32),
                       pltpu.VMEM((CHK,), jnp.float32),
                       pltpu.VMEM((NND,), jnp.float32),
                       pltpu.VMEM((NND,), jnp.float32),
                       pltpu.VMEM((NND,), jnp.float32),
                       pltpu.VMEM((NND,), jnp.float32),
                       pltpu.SemaphoreType.DMA],
    )(tx, ty, tz, idxd3)


BLKN = 2000


def _node_body(h_ref, pos_ref, R_ref, px_ref, py_ref, pz_ref, pc_ref,
               We2_ref, be2_ref, Wn1a_ref, Wn1b_ref, bn1_ref,
               Wn2_ref, bn2_ref, h_out_ref, pos_out_ref):
    R = R_ref[0] + R_ref[1]
    psx = jnp.sum(px_ref[...], axis=1, keepdims=True)
    psy = jnp.sum(py_ref[...], axis=1, keepdims=True)
    psz = jnp.sum(pz_ref[...], axis=1, keepdims=True)
    cnt = jnp.sum(pc_ref[...], axis=1, keepdims=True)
    agg = _hdot(R, _bf(We2_ref[...])) + cnt * be2_ref[...]
    h = h_ref[...]
    mid = jax.nn.relu(_bdot(h, Wn1a_ref[...])
                      + _bdot(agg, Wn1b_ref[...])
                      + bn1_ref[...])
    h_out_ref[...] = h + _bdot(mid, Wn2_ref[...]) + bn2_ref[...]
    lane = jax.lax.broadcasted_iota(jnp.int32, (BLKN, 16), 1)
    row = (jax.lax.broadcasted_iota(jnp.int32, (BLKN, 16), 0)
           + pl.program_id(0) * BLKN)
    ps = (jnp.where(lane == 0, psx, 0.0) + jnp.where(lane == 1, psy, 0.0)
          + jnp.where(lane == 2, psz, 0.0))
    upd = ps / jnp.clip(cnt, 1.0, None)
    pos_out_ref[...] = pos_ref[...] + jnp.where(row >= NPROT, upd, 0.0)


def _node_update(h, pos, RO, pxt, pyt, pzt, pct, We2, be2,
                 Wn1a, Wn1b, bn1, Wn2, bn2):
    nblk = NND // BLKN
    full = lambda s: pl.BlockSpec(s, lambda i: (0,) * len(s))
    colw = pl.BlockSpec((BLKN, NWRK), lambda i: (i, 0))
    return pl.pallas_call(
        _node_body,
        grid=(nblk,),
        in_specs=[
            pl.BlockSpec((BLKN, HID), lambda i: (i, 0)),
            pl.BlockSpec((BLKN, 16), lambda i: (i, 0)),
            pl.BlockSpec((2, BLKN, HID), lambda i: (0, i, 0)),
            colw, colw, colw, colw,
            full((HID, HID)), full((1, HID)),
            full((HID, HID)), full((HID, HID)), full((1, HID)),
            full((HID, HID)), full((1, HID)),
        ],
        out_specs=[pl.BlockSpec((BLKN, HID), lambda i: (i, 0)),
                   pl.BlockSpec((BLKN, 16), lambda i: (i, 0))],
        out_shape=[jax.ShapeDtypeStruct((NND, HID), jnp.float32),
                   jax.ShapeDtypeStruct((NND, 16), jnp.float32)],
    )(h, pos, RO, pxt, pyt, pzt, pct, We2, be2, Wn1a, Wn1b, bn1, Wn2, bn2)


def _head_body(h_ref, pos_ref, Wv_ref, bv_ref, out_ref):
    v = _bdot(h_ref[...], Wv_ref[...]) + bv_ref[...]
    lane = jax.lax.broadcasted_iota(jnp.int32, (NLIG, 16), 1)
    pos3 = jnp.where(lane < 3, pos_ref[...], 0.0)
    out_ref[...] = pos3 + v


def _head(h_l, pos_l, Wv_pad, bv_pad):
    return pl.pallas_call(
        _head_body,
        out_shape=jax.ShapeDtypeStruct((NLIG, 16), jnp.float32),
    )(h_l, pos_l, Wv_pad, bv_pad)


def kernel(protein_pos, protein_v, ligand_pos, ligand_v, batch_protein,
           batch_ligand, time_step, edge_index, W_p, b_p, W_l, b_l,
           Wt1, bt1, Wt2, bt2, We1, be1, We2, be2, Wn1, bn1, Wn2, bn2,
           Wc, bc, Wv, bv):
    f32 = jnp.float32
    ppos = jnp.pad(protein_pos, ((0, 0), (0, 13)))
    lpos = jnp.pad(ligand_pos, ((0, 0), (0, 13)))
    pv = jnp.pad(protein_v, ((0, 0), (0, 32 - PFD)))
    Wp_pad = jnp.pad(W_p, ((0, 32 - PFD), (0, 0)))
    lv = jnp.pad(ligand_v, ((0, 0), (0, 16 - LFD)))
    Wl_pad = jnp.pad(W_l, ((0, 16 - LFD), (0, 0)))
    half = HID // 2
    freq = jnp.exp(jnp.arange(half, dtype=f32)
                   * (-np.log(10000.0) / (half - 1)))[None, :]
    bp2 = batch_protein.astype(jnp.int32)[:, None]
    bl2 = batch_ligand.astype(jnp.int32)[:, None]
    t2 = time_step.astype(jnp.int32)[:, None]

    h, pos = _prologue(ppos, pv, lpos, lv, bp2, bl2, t2, freq,
                       Wp_pad, b_p[None, :], Wl_pad, b_l[None, :],
                       Wt1, bt1[None, :], Wt2, bt2[None, :])

    idxs3 = edge_index[0].astype(jnp.int32).reshape(NWRK, NCHK, CHK)
    idxd3 = edge_index[1].astype(jnp.int32).reshape(NWRK, NCHK, CHK)

    for i in range(NLAY):
        ta, tb = _prep(h, We1[i, :HID], We1[i, HID:2 * HID], be1[i][None, :])
        px, py, pz = pos[:, 0], pos[:, 1], pos[:, 2]
        ga, gb, rx, ry, rz, d2 = _gather_stage(ta, tb, px, py, pz,
                                               idxd3, idxs3)
        r, tx, ty, tz = _edge_stage(
            ga, gb, rx[:, None], ry[:, None], rz[:, None], d2[:, None],
            We1[i, 2 * HID][None, :], We2[i], be2[i][None, :],
            Wc[i, :, 0][None, :], bc[i][None, :])
        RO = _scatter_r_stage(r, idxd3)
        pxt, pyt, pzt, pct = _scatter_p_stage(
            tx[:, 0], ty[:, 0], tz[:, 0], idxd3)
        h, pos = _node_update(h, pos, RO[:, :NND], pxt.T, pyt.T, pzt.T,
                              pct.T,
                              We2[i], be2[i][None, :],
                              Wn1[i, :HID], Wn1[i, HID:], bn1[i][None, :],
                              Wn2[i], bn2[i][None, :])

    Wv_pad = jnp.pad(Wv, ((0, 0), (3, 0)))
    bv_pad = jnp.pad(bv, ((3, 0),))[None, :]
    return _head(h[NPROT:], pos[NPROT:], Wv_pad, bv_pad)


# speedup vs baseline: 3.9631x; 1.7979x over previous
import functools

import jax
import jax.numpy as jnp
import numpy as np
from jax.experimental import pallas as pl
from jax.experimental.pallas import tpu as pltpu
from jax.experimental.pallas import tpu_sc as plsc

NPROT, NLIG, NB = 8000, 2000, 16
NND = NPROT + NLIG
EDG, HID, NLAY = 320000, 128, 4
PFD, LFD = 27, 13
BLK_E = 3200
BLKN = 2000

SC_NC, SC_NS, SC_L = 2, 16, 16
NWRK = SC_NC * SC_NS
EPW = EDG // NWRK
CHK = 80
NCHK = EPW // CHK
NPAD = 10240
ROWS_PT = NPAD // SC_NS


def _bdot(a, b):
    return jnp.dot(a.astype(jnp.bfloat16), b.astype(jnp.bfloat16),
                   preferred_element_type=jnp.float32)


def _bf(x):
    return x.astype(jnp.bfloat16).astype(jnp.float32)


_hdot = functools.partial(jnp.dot, preferred_element_type=jnp.float32,
                          precision=jax.lax.Precision.HIGHEST)


def _prologue_body(ppos_ref, pv_ref, lpos_ref, lv_ref, bp_ref, bl_ref, t_ref,
                   freq_ref, Wp_ref, bp_w_ref, Wl_ref, bl_w_ref,
                   Wt1_ref, bt1_ref, Wt2_ref, bt2_ref,
                   Wa_ref, Wb_ref, be1_ref,
                   h_ref, pos_ref, ta_ref, tb_ref):
    iota_b = jax.lax.broadcasted_iota(jnp.int32, (1, NB), 1)
    oh_p = (bp_ref[...] == iota_b).astype(jnp.float32)
    oh_l = (bl_ref[...] == iota_b).astype(jnp.float32)
    ppos = ppos_ref[...]
    sums = _hdot(oh_p.T, ppos)
    cnts = jnp.sum(oh_p, axis=0, keepdims=True)
    offset = sums / jnp.clip(cnts.T, 1.0, None)
    ppos = ppos - _hdot(oh_p, offset)
    lpos = lpos_ref[...] - _hdot(oh_l, offset)
    t = t_ref[...].astype(jnp.float32)
    arg = t * freq_ref[...]
    emb = jnp.concatenate([jnp.sin(arg), jnp.cos(arg)], axis=1)
    temb = _bdot(jax.nn.relu(_bdot(emb, Wt1_ref[...]) + bt1_ref[...]),
                 Wt2_ref[...]) + bt2_ref[...]
    h_p = _bdot(pv_ref[...], Wp_ref[...]) + bp_w_ref[...]
    h_l = (_bdot(lv_ref[...], Wl_ref[...]) + bl_w_ref[...]
           + _hdot(oh_l, temb))
    h = jnp.concatenate([h_p, h_l], axis=0)
    h_ref[...] = h
    pos_ref[:NPROT, :] = ppos
    pos_ref[NPROT:, :] = lpos
    ta_ref[...] = _bdot(h, Wa_ref[...]) + be1_ref[...]
    tb_ref[...] = _bdot(h, Wb_ref[...])


def _prologue(ppos, pv, lpos, lv, bp, bl, t, freq, Wp, bpw, Wl, blw,
              Wt1, bt1, Wt2, bt2, Wa, Wb, be1):
    return pl.pallas_call(
        _prologue_body,
        out_shape=[jax.ShapeDtypeStruct((NND, HID), jnp.float32),
                   jax.ShapeDtypeStruct((NND, 16), jnp.float32),
                   jax.ShapeDtypeStruct((NND, HID), jnp.float32),
                   jax.ShapeDtypeStruct((NND, HID), jnp.float32)],
    )(ppos, pv, lpos, lv, bp, bl, t, freq, Wp, bpw, Wl, blw,
      Wt1, bt1, Wt2, bt2, Wa, Wb, be1)


def _sc_mesh():
    return plsc.VectorSubcoreMesh(core_axis_name="c", subcore_axis_name="s")


def _gather_body(ta_hbm, tb_hbm, px_hbm, py_hbm, pz_hbm, w2h_hbm,
                 idxd_hbm, idxs_hbm, u_hbm,
                 idxd_v, idxs_v, ba0, bb0, ba1, bb1, px_v, py_v, pz_v,
                 w2h_v, d2_v, gs0, gs1, ws0, ws1):
    cid = jax.lax.axis_index("c")
    sid = jax.lax.axis_index("s")
    wid = cid * SC_NS + sid
    pltpu.sync_copy(idxd_hbm.at[wid], idxd_v)
    pltpu.sync_copy(idxs_hbm.at[wid], idxs_v)
    pltpu.sync_copy(px_hbm, px_v)
    pltpu.sync_copy(py_hbm, py_v)
    pltpu.sync_copy(pz_hbm, pz_v)
    pltpu.sync_copy(w2h_hbm, w2h_v)

    bufs = ((ba0, bb0, gs0, ws0), (ba1, bb1, gs1, ws1))

    def issue(c, b):
        ba, bb, gs, _ = bufs[b]
        pltpu.async_copy(ta_hbm.at[idxd_v.at[c]], ba, gs)
        pltpu.async_copy(tb_hbm.at[idxs_v.at[c]], bb, gs)

    def process(c, b):
        ba, bb, gs, ws = bufs[b]
        pltpu.make_async_copy(ta_hbm.at[pl.ds(0, CHK)], ba, gs).wait()
        pltpu.make_async_copy(tb_hbm.at[pl.ds(0, CHK)], bb, gs).wait()

        def jstep(j, _):
            id_d = idxd_v[c, pl.ds(j * SC_L, SC_L)]
            id_s = idxs_v[c, pl.ds(j * SC_L, SC_L)]
            relx = (plsc.load_gather(px_v, [id_s])
                    - plsc.load_gather(px_v, [id_d]))
            rely = (plsc.load_gather(py_v, [id_s])
                    - plsc.load_gather(py_v, [id_d]))
            relz = (plsc.load_gather(pz_v, [id_s])
                    - plsc.load_gather(pz_v, [id_d]))
            d2 = relx * relx + rely * rely + relz * relz
            bits = jax.lax.bitcast_convert_type(d2, jnp.int32)
            lsb = jax.lax.shift_right_logical(bits, 16) & 1
            bits = (bits + 32767 + lsb) & jnp.int32(-65536)
            d2_v[pl.ds(j * SC_L, SC_L)] = jax.lax.bitcast_convert_type(
                bits, jnp.float32)
            return ()

        jax.lax.fori_loop(0, CHK // SC_L, jstep, ())

        def estep(e, _):
            e_vec = jnp.full((SC_L,), e, jnp.int32)
            d2s = plsc.load_gather(d2_v, [e_vec])
            for f in range(HID // SC_L):
                sl = pl.ds(f * SC_L, SC_L)
                ba[e, sl] = (ba[e, sl] + bb[e, sl]) + d2s * w2h_v[sl]
            return ()

        jax.lax.fori_loop(0, CHK, estep, ())
        pltpu.async_copy(ba, u_hbm.at[pl.ds(wid * EPW + c * CHK, CHK)], ws)
        pltpu.make_async_copy(ba, u_hbm.at[pl.ds(0, CHK)], ws).wait()

    issue(0, 0)
    issue(1, 1)

    def pair(i, _):
        for b in range(2):
            c = 2 * i + b

            @pl.when(c < NCHK)
            def _():
                process(c, b)

                @pl.when(c + 2 < NCHK)
                def _():
                    issue(c + 2, b)
        return ()

    jax.lax.fori_loop(0, (NCHK + 1) // 2, pair, ())


def _gather_stage(ta, tb, px, py, pz, w2hb, idxd3, idxs3):
    return pl.kernel(
        _gather_body,
        mesh=_sc_mesh(),
        compiler_params=pltpu.CompilerParams(needs_layout_passes=False),
        out_type=jax.ShapeDtypeStruct((EDG, HID), jnp.float32),
        scratch_types=[pltpu.VMEM((NCHK, CHK), jnp.int32),
                       pltpu.VMEM((NCHK, CHK), jnp.int32),
                       pltpu.VMEM((CHK, HID), jnp.float32),
                       pltpu.VMEM((CHK, HID), jnp.float32),
                       pltpu.VMEM((CHK, HID), jnp.float32),
                       pltpu.VMEM((CHK, HID), jnp.float32),
                       pltpu.VMEM((NND,), jnp.float32),
                       pltpu.VMEM((NND,), jnp.float32),
                       pltpu.VMEM((NND,), jnp.float32),
                       pltpu.VMEM((HID,), jnp.float32),
                       pltpu.VMEM((CHK,), jnp.float32),
                       pltpu.SemaphoreType.DMA,
                       pltpu.SemaphoreType.DMA,
                       pltpu.SemaphoreType.DMA,
                       pltpu.SemaphoreType.DMA],
    )(ta, tb, px, py, pz, w2hb, idxd3, idxs3)


def _edge_body(u_ref, We2_ref, be2_ref, wc_ref, bc_ref, r_ref, coef_ref):
    r = _bf(jax.nn.relu(u_ref[...]))
    r_ref[...] = r
    ef = _bdot(r, We2_ref[...]) + be2_ref[...]
    dot = (jnp.sum(_bf(ef) * _bf(wc_ref[...]), axis=1, keepdims=True)
           + bc_ref[0, 0])
    coef_ref[...] = jnp.tanh(dot)


def _edge_stage(u, We2, be2, wc, bc):
    nblk = EDG // BLK_E
    return pl.pallas_call(
        _edge_body,
        grid=(nblk,),
        in_specs=[
            pl.BlockSpec((BLK_E, HID), lambda i: (i, 0)),
            pl.BlockSpec((HID, HID), lambda i: (0, 0)),
            pl.BlockSpec((1, HID), lambda i: (0, 0)),
            pl.BlockSpec((1, HID), lambda i: (0, 0)),
            pl.BlockSpec((1, 1), lambda i: (0, 0), memory_space=pltpu.SMEM),
        ],
        out_specs=[pl.BlockSpec((BLK_E, HID), lambda i: (i, 0)),
                   pl.BlockSpec((BLK_E, 1), lambda i: (i, 0))],
        out_shape=[jax.ShapeDtypeStruct((EDG, HID), jnp.float32),
                   jax.ShapeDtypeStruct((EDG, 1), jnp.float32)],
    )(u, We2, be2, wc, bc)


def _scatter_r_body(r_hbm, idxd_hbm, ro_hbm,
                    idxd_v, br0, br1, zr, accr, ls0, ls1, as0, as1):
    cid = jax.lax.axis_index("c")
    sid = jax.lax.axis_index("s")
    wid = cid * SC_NS + sid
    pltpu.sync_copy(idxd_hbm.at[wid], idxd_v)

    zeros16 = jnp.zeros((SC_L,), jnp.float32)

    def zrow(i, _):
        def zcol(j, _):
            zr[i, pl.ds(j * SC_L, SC_L)] = zeros16
            return ()
        jax.lax.fori_loop(0, HID // SC_L, zcol, ())
        return ()

    jax.lax.fori_loop(0, SC_L, zrow, ())

    def zshared(k, _):
        pltpu.sync_copy(zr, accr.at[pl.ds(sid * ROWS_PT + k * SC_L, SC_L)])
        return ()

    jax.lax.fori_loop(0, ROWS_PT // SC_L, zshared, ())
    plsc.subcore_barrier()

    bufs = ((br0, ls0, as0), (br1, ls1, as1))

    def issue(c, b):
        br, ls, _ = bufs[b]
        pltpu.async_copy(r_hbm.at[pl.ds(wid * EPW + c * CHK, CHK)], br, ls)

    def process(c, b):
        br, ls, asem = bufs[b]
        pltpu.make_async_copy(r_hbm.at[pl.ds(0, CHK)], br, ls).wait()
        pltpu.async_copy(br, accr.at[idxd_v.at[c]], asem, add=True)
        pltpu.make_async_copy(br, accr.at[idxd_v.at[0]], asem).wait()

    issue(0, 0)
    issue(1, 1)

    def pair(i, _):
        for b in range(2):
            c = 2 * i + b

            @pl.when(c < NCHK)
            def _():
                process(c, b)

                @pl.when(c + 2 < NCHK)
                def _():
                    issue(c + 2, b)
        return ()

    jax.lax.fori_loop(0, (NCHK + 1) // 2, pair, ())
    plsc.subcore_barrier()

    out0 = sid * ROWS_PT
    pltpu.sync_copy(accr.at[pl.ds(out0, ROWS_PT)],
                    ro_hbm.at[cid, pl.ds(out0, ROWS_PT)])


def _scatter_r_stage(r, idxd3):
    return pl.kernel(
        _scatter_r_body,
        mesh=_sc_mesh(),
        compiler_params=pltpu.CompilerParams(needs_layout_passes=False),
        out_type=jax.ShapeDtypeStruct((SC_NC, NPAD, HID), jnp.float32),
        scratch_types=[pltpu.VMEM((NCHK, CHK), jnp.int32),
                       pltpu.VMEM((CHK, HID), jnp.float32),
                       pltpu.VMEM((CHK, HID), jnp.float32),
                       pltpu.VMEM((SC_L, HID), jnp.float32),
                       pltpu.VMEM_SHARED((NPAD, HID), jnp.float32),
                       pltpu.SemaphoreType.DMA,
                       pltpu.SemaphoreType.DMA,
                       pltpu.SemaphoreType.DMA,
                       pltpu.SemaphoreType.DMA],
    )(r, idxd3)


def _scatter_p_body(cf_hbm, idxd_hbm, idxs_hbm, px_hbm, py_hbm, pz_hbm,
                    px_o, py_o, pz_o, pc_o,
                    idxd_v, idxs_v, bufc, pxv, pyv, pzv,
                    accx, accy, accz, accc, sem):
    cid = jax.lax.axis_index("c")
    sid = jax.lax.axis_index("s")
    wid = cid * SC_NS + sid
    pltpu.sync_copy(idxd_hbm.at[wid], idxd_v)
    pltpu.sync_copy(idxs_hbm.at[wid], idxs_v)
    pltpu.sync_copy(cf_hbm.at[pl.ds(wid * EPW, EPW)], bufc)
    pltpu.sync_copy(px_hbm, pxv)
    pltpu.sync_copy(py_hbm, pyv)
    pltpu.sync_copy(pz_hbm, pzv)

    zeros16 = jnp.zeros((SC_L,), jnp.float32)

    def zacc(k, _):
        sl = pl.ds(k * SC_L, SC_L)
        accx[sl] = zeros16
        accy[sl] = zeros16
        accz[sl] = zeros16
        accc[sl] = zeros16
        return ()

    jax.lax.fori_loop(0, NND // SC_L, zacc, ())

    ones16 = jnp.full((SC_L,), 1.0, jnp.float32)

    def jstep(j, _):
        sl = pl.ds(j * SC_L, SC_L)
        idv = idxd_v[sl]
        isv = idxs_v[sl]
        cf = bufc[sl]
        trx = (plsc.load_gather(pxv, [isv]) - plsc.load_gather(pxv, [idv])) * cf
        try_ = (plsc.load_gather(pyv, [isv]) - plsc.load_gather(pyv, [idv])) * cf
        trz = (plsc.load_gather(pzv, [isv]) - plsc.load_gather(pzv, [idv])) * cf
        plsc.addupdate_scatter(accx, [idv], trx)
        plsc.addupdate_scatter(accy, [idv], try_)
        plsc.addupdate_scatter(accz, [idv], trz)
        plsc.addupdate_scatter(accc, [idv], ones16)
        return ()

    jax.lax.fori_loop(0, EPW // SC_L, jstep, ())

    pltpu.sync_copy(accx, px_o.at[wid])
    pltpu.sync_copy(accy, py_o.at[wid])
    pltpu.sync_copy(accz, pz_o.at[wid])
    pltpu.sync_copy(accc, pc_o.at[wid])


def _scatter_p_stage(coef, idxd2, idxs2, px, py, pz):
    return pl.kernel(
        _scatter_p_body,
        mesh=_sc_mesh(),
        compiler_params=pltpu.CompilerParams(needs_layout_passes=False),
        out_type=[jax.ShapeDtypeStruct((NWRK, NND), jnp.float32),
                  jax.ShapeDtypeStruct((NWRK, NND), jnp.float32),
                  jax.ShapeDtypeStruct((NWRK, NND), jnp.float32),
                  jax.ShapeDtypeStruct((NWRK, NND), jnp.float32)],
        scratch_types=[pltpu.VMEM((EPW,), jnp.int32),
                       pltpu.VMEM((EPW,), jnp.int32),
                       pltpu.VMEM((EPW,), jnp.float32),
                       pltpu.VMEM((NND,), jnp.float32),
                       pltpu.VMEM((NND,), jnp.float32),
                       pltpu.VMEM((NND,), jnp.float32),
                       pltpu.VMEM((NND,), jnp.float32),
                       pltpu.VMEM((NND,), jnp.float32),
                       pltpu.VMEM((NND,), jnp.float32),
                       pltpu.VMEM((NND,), jnp.float32),
                       pltpu.SemaphoreType.DMA],
    )(coef, idxd2, idxs2, px, py, pz)


def _node_body(h_ref, pos_ref, R_ref, px_ref, py_ref, pz_ref, pc_ref,
               We2_ref, be2_ref, Wn1a_ref, Wn1b_ref, bn1_ref,
               Wn2_ref, bn2_ref, Wa_ref, Wb_ref, be1_ref,
               h_out_ref, pos_out_ref, ta_ref, tb_ref):
    R = R_ref[0] + R_ref[1]
    psx = jnp.sum(px_ref[...], axis=1, keepdims=True)
    psy = jnp.sum(py_ref[...], axis=1, keepdims=True)
    psz = jnp.sum(pz_ref[...], axis=1, keepdims=True)
    cnt = jnp.sum(pc_ref[...], axis=1, keepdims=True)
    agg = _hdot(R, _bf(We2_ref[...])) + cnt * be2_ref[...]
    h = h_ref[...]
    mid = jax.nn.relu(_bdot(h, Wn1a_ref[...])
                      + _bdot(agg, Wn1b_ref[...])
                      + bn1_ref[...])
    h_new = h + _bdot(mid, Wn2_ref[...]) + bn2_ref[...]
    h_out_ref[...] = h_new
    lane = jax.lax.broadcasted_iota(jnp.int32, (BLKN, 16), 1)
    row = (jax.lax.broadcasted_iota(jnp.int32, (BLKN, 16), 0)
           + pl.program_id(0) * BLKN)
    ps = (jnp.where(lane == 0, psx, 0.0) + jnp.where(lane == 1, psy, 0.0)
          + jnp.where(lane == 2, psz, 0.0))
    upd = ps / jnp.clip(cnt, 1.0, None)
    pos_out_ref[...] = pos_ref[...] + jnp.where(row >= NPROT, upd, 0.0)
    ta_ref[...] = _bdot(h_new, Wa_ref[...]) + be1_ref[...]
    tb_ref[...] = _bdot(h_new, Wb_ref[...])


def _node_update(h, pos, RO, pxt, pyt, pzt, pct, We2, be2,
                 Wn1a, Wn1b, bn1, Wn2, bn2, Wa, Wb, be1):
    nblk = NND // BLKN
    full = lambda s: pl.BlockSpec(s, lambda i: (0,) * len(s))
    colw = pl.BlockSpec((BLKN, NWRK), lambda i: (i, 0))
    rowb = pl.BlockSpec((BLKN, HID), lambda i: (i, 0))
    return pl.pallas_call(
        _node_body,
        grid=(nblk,),
        in_specs=[
            rowb,
            pl.BlockSpec((BLKN, 16), lambda i: (i, 0)),
            pl.BlockSpec((2, BLKN, HID), lambda i: (0, i, 0)),
            colw, colw, colw, colw,
            full((HID, HID)), full((1, HID)),
            full((HID, HID)), full((HID, HID)), full((1, HID)),
            full((HID, HID)), full((1, HID)),
            full((HID, HID)), full((HID, HID)), full((1, HID)),
        ],
        out_specs=[rowb,
                   pl.BlockSpec((BLKN, 16), lambda i: (i, 0)),
                   rowb, rowb],
        out_shape=[jax.ShapeDtypeStruct((NND, HID), jnp.float32),
                   jax.ShapeDtypeStruct((NND, 16), jnp.float32),
                   jax.ShapeDtypeStruct((NND, HID), jnp.float32),
                   jax.ShapeDtypeStruct((NND, HID), jnp.float32)],
    )(h, pos, RO, pxt, pyt, pzt, pct, We2, be2, Wn1a, Wn1b, bn1, Wn2, bn2,
      Wa, Wb, be1)


def _head_body(h_ref, pos_ref, Wv_ref, bv_ref, out_ref):
    v = _bdot(h_ref[...], Wv_ref[...]) + bv_ref[...]
    lane = jax.lax.broadcasted_iota(jnp.int32, (NLIG, 16), 1)
    pos3 = jnp.where(lane < 3, pos_ref[...], 0.0)
    out_ref[...] = pos3 + v


def _head(h_l, pos_l, Wv_pad, bv_pad):
    return pl.pallas_call(
        _head_body,
        out_shape=jax.ShapeDtypeStruct((NLIG, 16), jnp.float32),
    )(h_l, pos_l, Wv_pad, bv_pad)


def kernel(protein_pos, protein_v, ligand_pos, ligand_v, batch_protein,
           batch_ligand, time_step, edge_index, W_p, b_p, W_l, b_l,
           Wt1, bt1, Wt2, bt2, We1, be1, We2, be2, Wn1, bn1, Wn2, bn2,
           Wc, bc, Wv, bv):
    f32 = jnp.float32
    ppos = jnp.pad(protein_pos, ((0, 0), (0, 13)))
    lpos = jnp.pad(ligand_pos, ((0, 0), (0, 13)))
    pv = jnp.pad(protein_v, ((0, 0), (0, 32 - PFD)))
    Wp_pad = jnp.pad(W_p, ((0, 32 - PFD), (0, 0)))
    lv = jnp.pad(ligand_v, ((0, 0), (0, 16 - LFD)))
    Wl_pad = jnp.pad(W_l, ((0, 16 - LFD), (0, 0)))
    half = HID // 2
    freq = jnp.exp(jnp.arange(half, dtype=f32)
                   * (-np.log(10000.0) / (half - 1)))[None, :]
    bp2 = batch_protein.astype(jnp.int32)[:, None]
    bl2 = batch_ligand.astype(jnp.int32)[:, None]
    t2 = time_step.astype(jnp.int32)[:, None]

    h, pos, ta, tb = _prologue(
        ppos, pv, lpos, lv, bp2, bl2, t2, freq,
        Wp_pad, b_p[None, :], Wl_pad, b_l[None, :],
        Wt1, bt1[None, :], Wt2, bt2[None, :],
        We1[0, :HID], We1[0, HID:2 * HID], be1[0][None, :])

    src_i = edge_index[0].astype(jnp.int32)
    dst_i = edge_index[1].astype(jnp.int32)
    idxs3 = src_i.reshape(NWRK, NCHK, CHK)
    idxd3 = dst_i.reshape(NWRK, NCHK, CHK)
    idxs2 = src_i.reshape(NWRK, EPW)
    idxd2 = dst_i.reshape(NWRK, EPW)

    for i in range(NLAY):
        px, py, pz = pos[:, 0], pos[:, 1], pos[:, 2]
        u = _gather_stage(ta, tb, px, py, pz, _bf(We1[i, 2 * HID]),
                          idxd3, idxs3)
        r, coef = _edge_stage(u, We2[i], be2[i][None, :],
                              Wc[i, :, 0][None, :], bc[i][None, :])
        RO = _scatter_r_stage(r, idxd3)
        pxt, pyt, pzt, pct = _scatter_p_stage(coef[:, 0], idxd2, idxs2,
                                              px, py, pz)
        j = (i + 1) % NLAY
        h, pos, ta, tb = _node_update(
            h, pos, RO[:, :NND], pxt.T, pyt.T, pzt.T, pct.T,
            We2[i], be2[i][None, :], Wn1[i, :HID], Wn1[i, HID:],
            bn1[i][None, :], Wn2[i], bn2[i][None, :],
            We1[j, :HID], We1[j, HID:2 * HID], be1[j][None, :])

    Wv_pad = jnp.pad(Wv, ((0, 0), (3, 0)))
    bv_pad = jnp.pad(bv, ((3, 0),))[None, :]
    return _head(h[NPROT:], pos[NPROT:], Wv_pad, bv_pad)


# speedup vs baseline: 6.4710x; 1.6328x over previous
import functools

import jax
import jax.numpy as jnp
import numpy as np
from jax.experimental import pallas as pl
from jax.experimental.pallas import tpu as pltpu
from jax.experimental.pallas import tpu_sc as plsc

NPROT, NLIG, NB = 8000, 2000, 16
NND = NPROT + NLIG
EDG, HID, NLAY = 320000, 128, 4
PFD, LFD = 27, 13
BLK_E = 3200
BLKN = 2000

SC_NC, SC_NS, SC_L = 2, 16, 16
NWRK = SC_NC * SC_NS
EPW = EDG // NWRK
CHK = 80
NCHK = EPW // CHK
NPAD = 10240
ROWS_PT = NPAD // SC_NS


def _bdot(a, b):
    return jnp.dot(a.astype(jnp.bfloat16), b.astype(jnp.bfloat16),
                   preferred_element_type=jnp.float32)


def _bf(x):
    return x.astype(jnp.bfloat16).astype(jnp.float32)


_hdot = functools.partial(jnp.dot, preferred_element_type=jnp.float32,
                          precision=jax.lax.Precision.HIGHEST)


def _prologue_body(ppos_ref, pv_ref, lpos_ref, lv_ref, bp_ref, bl_ref, t_ref,
                   freq_ref, Wp_ref, bp_w_ref, Wl_ref, bl_w_ref,
                   Wt1_ref, bt1_ref, Wt2_ref, bt2_ref,
                   Wa_ref, Wb_ref, be1_ref,
                   h_ref, pos_ref, ta_ref, tb_ref):
    iota_b = jax.lax.broadcasted_iota(jnp.int32, (1, NB), 1)
    oh_p = (bp_ref[...] == iota_b).astype(jnp.float32)
    oh_l = (bl_ref[...] == iota_b).astype(jnp.float32)
    ppos = ppos_ref[...]
    sums = _hdot(oh_p.T, ppos)
    cnts = jnp.sum(oh_p, axis=0, keepdims=True)
    offset = sums / jnp.clip(cnts.T, 1.0, None)
    ppos = ppos - _hdot(oh_p, offset)
    lpos = lpos_ref[...] - _hdot(oh_l, offset)
    t = t_ref[...].astype(jnp.float32)
    arg = t * freq_ref[...]
    emb = jnp.concatenate([jnp.sin(arg), jnp.cos(arg)], axis=1)
    temb = _bdot(jax.nn.relu(_bdot(emb, Wt1_ref[...]) + bt1_ref[...]),
                 Wt2_ref[...]) + bt2_ref[...]
    h_p = _bdot(pv_ref[...], Wp_ref[...]) + bp_w_ref[...]
    h_l = (_bdot(lv_ref[...], Wl_ref[...]) + bl_w_ref[...]
           + _hdot(oh_l, temb))
    h = jnp.concatenate([h_p, h_l], axis=0)
    h_ref[...] = h
    pos_ref[:NPROT, :] = ppos
    pos_ref[NPROT:, :] = lpos
    ta_ref[...] = _bdot(h, Wa_ref[...]) + be1_ref[...]
    tb_ref[...] = _bdot(h, Wb_ref[...])


def _prologue(ppos, pv, lpos, lv, bp, bl, t, freq, Wp, bpw, Wl, blw,
              Wt1, bt1, Wt2, bt2, Wa, Wb, be1):
    return pl.pallas_call(
        _prologue_body,
        out_shape=[jax.ShapeDtypeStruct((NND, HID), jnp.float32),
                   jax.ShapeDtypeStruct((NND, 16), jnp.float32),
                   jax.ShapeDtypeStruct((NND, HID), jnp.float32),
                   jax.ShapeDtypeStruct((NND, HID), jnp.float32)],
    )(ppos, pv, lpos, lv, bp, bl, t, freq, Wp, bpw, Wl, blw,
      Wt1, bt1, Wt2, bt2, Wa, Wb, be1)


def _sc_mesh():
    return plsc.VectorSubcoreMesh(core_axis_name="c", subcore_axis_name="s")


def _gather_body(ta_hbm, tb_hbm, px_hbm, py_hbm, pz_hbm, w2h_hbm,
                 idxd_hbm, idxs_hbm, u_hbm,
                 idxd_v, idxs_v, ba0, bb0, ba1, bb1, px_v, py_v, pz_v,
                 w2h_v, d2_v, gs0, gs1, ws0, ws1):
    cid = jax.lax.axis_index("c")
    sid = jax.lax.axis_index("s")
    wid = cid * SC_NS + sid
    pltpu.sync_copy(idxd_hbm.at[wid], idxd_v)
    pltpu.sync_copy(idxs_hbm.at[wid], idxs_v)
    pltpu.sync_copy(px_hbm, px_v)
    pltpu.sync_copy(py_hbm, py_v)
    pltpu.sync_copy(pz_hbm, pz_v)
    pltpu.sync_copy(w2h_hbm, w2h_v)

    bufs = ((ba0, bb0, gs0, ws0), (ba1, bb1, gs1, ws1))

    def issue(c, b):
        ba, bb, gs, _ = bufs[b]
        pltpu.async_copy(ta_hbm.at[idxd_v.at[c]], ba, gs)
        pltpu.async_copy(tb_hbm.at[idxs_v.at[c]], bb, gs)

    def process(c, b):
        ba, bb, gs, ws = bufs[b]
        pltpu.make_async_copy(ta_hbm.at[pl.ds(0, CHK)], ba, gs).wait()
        pltpu.make_async_copy(tb_hbm.at[pl.ds(0, CHK)], bb, gs).wait()

        @functools.partial(plsc.parallel_loop, 0, CHK // SC_L, unroll=5)
        def jstep(j):
            id_d = idxd_v[c, pl.ds(j * SC_L, SC_L)]
            id_s = idxs_v[c, pl.ds(j * SC_L, SC_L)]
            relx = (plsc.load_gather(px_v, [id_s])
                    - plsc.load_gather(px_v, [id_d]))
            rely = (plsc.load_gather(py_v, [id_s])
                    - plsc.load_gather(py_v, [id_d]))
            relz = (plsc.load_gather(pz_v, [id_s])
                    - plsc.load_gather(pz_v, [id_d]))
            d2 = relx * relx + rely * rely + relz * relz
            bits = jax.lax.bitcast_convert_type(d2, jnp.int32)
            lsb = jax.lax.shift_right_logical(bits, 16) & 1
            bits = (bits + 32767 + lsb) & jnp.int32(-65536)
            d2_v[pl.ds(j * SC_L, SC_L)] = jax.lax.bitcast_convert_type(
                bits, jnp.float32)

        @functools.partial(plsc.parallel_loop, 0, CHK, unroll=4)
        def estep(e):
            e_vec = jnp.full((SC_L,), e, jnp.int32)
            d2s = plsc.load_gather(d2_v, [e_vec])
            for f in range(HID // SC_L):
                sl = pl.ds(f * SC_L, SC_L)
                ba[e, sl] = (ba[e, sl] + bb[e, sl]) + d2s * w2h_v[sl]
        pltpu.async_copy(ba, u_hbm.at[pl.ds(wid * EPW + c * CHK, CHK)], ws)
        pltpu.make_async_copy(ba, u_hbm.at[pl.ds(0, CHK)], ws).wait()

    issue(0, 0)
    issue(1, 1)

    def pair(i, _):
        for b in range(2):
            c = 2 * i + b

            @pl.when(c < NCHK)
            def _():
                process(c, b)

                @pl.when(c + 2 < NCHK)
                def _():
                    issue(c + 2, b)
        return ()

    jax.lax.fori_loop(0, (NCHK + 1) // 2, pair, ())


def _gather_stage(ta, tb, px, py, pz, w2hb, idxd3, idxs3):
    return pl.kernel(
        _gather_body,
        mesh=_sc_mesh(),
        compiler_params=pltpu.CompilerParams(needs_layout_passes=False),
        out_type=jax.ShapeDtypeStruct((EDG, HID), jnp.float32),
        scratch_types=[pltpu.VMEM((NCHK, CHK), jnp.int32),
                       pltpu.VMEM((NCHK, CHK), jnp.int32),
                       pltpu.VMEM((CHK, HID), jnp.float32),
                       pltpu.VMEM((CHK, HID), jnp.float32),
                       pltpu.VMEM((CHK, HID), jnp.float32),
                       pltpu.VMEM((CHK, HID), jnp.float32),
                       pltpu.VMEM((NND,), jnp.float32),
                       pltpu.VMEM((NND,), jnp.float32),
                       pltpu.VMEM((NND,), jnp.float32),
                       pltpu.VMEM((HID,), jnp.float32),
                       pltpu.VMEM((CHK,), jnp.float32),
                       pltpu.SemaphoreType.DMA,
                       pltpu.SemaphoreType.DMA,
                       pltpu.SemaphoreType.DMA,
                       pltpu.SemaphoreType.DMA],
    )(ta, tb, px, py, pz, w2hb, idxd3, idxs3)


def _edge_body(u_ref, We2_ref, be2_ref, wc_ref, bc_ref, r_ref, coef_ref):
    r = _bf(jax.nn.relu(u_ref[...]))
    r_ref[...] = r
    ef = _bdot(r, We2_ref[...]) + be2_ref[...]
    dot = (jnp.sum(_bf(ef) * _bf(wc_ref[...]), axis=1, keepdims=True)
           + bc_ref[0, 0])
    coef_ref[...] = jnp.tanh(dot)


def _edge_stage(u, We2, be2, wc, bc):
    nblk = EDG // BLK_E
    return pl.pallas_call(
        _edge_body,
        grid=(nblk,),
        in_specs=[
            pl.BlockSpec((BLK_E, HID), lambda i: (i, 0)),
            pl.BlockSpec((HID, HID), lambda i: (0, 0)),
            pl.BlockSpec((1, HID), lambda i: (0, 0)),
            pl.BlockSpec((1, HID), lambda i: (0, 0)),
            pl.BlockSpec((1, 1), lambda i: (0, 0), memory_space=pltpu.SMEM),
        ],
        out_specs=[pl.BlockSpec((BLK_E, HID), lambda i: (i, 0)),
                   pl.BlockSpec((BLK_E, 1), lambda i: (i, 0))],
        out_shape=[jax.ShapeDtypeStruct((EDG, HID), jnp.float32),
                   jax.ShapeDtypeStruct((EDG, 1), jnp.float32)],
    )(u, We2, be2, wc, bc)


def _scatter_r_body(r_hbm, idxd_hbm, ro_hbm,
                    idxd_v, br0, br1, zr, accr, ls0, ls1, as0, as1):
    cid = jax.lax.axis_index("c")
    sid = jax.lax.axis_index("s")
    wid = cid * SC_NS + sid
    pltpu.sync_copy(idxd_hbm.at[wid], idxd_v)

    zeros16 = jnp.zeros((SC_L,), jnp.float32)

    def zrow(i, _):
        def zcol(j, _):
            zr[i, pl.ds(j * SC_L, SC_L)] = zeros16
            return ()
        jax.lax.fori_loop(0, HID // SC_L, zcol, ())
        return ()

    jax.lax.fori_loop(0, SC_L, zrow, ())

    def zshared(k, _):
        pltpu.sync_copy(zr, accr.at[pl.ds(sid * ROWS_PT + k * SC_L, SC_L)])
        return ()

    jax.lax.fori_loop(0, ROWS_PT // SC_L, zshared, ())
    plsc.subcore_barrier()

    bufs = ((br0, ls0, as0), (br1, ls1, as1))

    def issue(c, b):
        br, ls, _ = bufs[b]
        pltpu.async_copy(r_hbm.at[pl.ds(wid * EPW + c * CHK, CHK)], br, ls)

    def process(c, b):
        br, ls, asem = bufs[b]
        pltpu.make_async_copy(r_hbm.at[pl.ds(0, CHK)], br, ls).wait()
        pltpu.async_copy(br, accr.at[idxd_v.at[c]], asem, add=True)
        pltpu.make_async_copy(br, accr.at[idxd_v.at[0]], asem).wait()

    issue(0, 0)
    issue(1, 1)

    def pair(i, _):
        for b in range(2):
            c = 2 * i + b

            @pl.when(c < NCHK)
            def _():
                process(c, b)

                @pl.when(c + 2 < NCHK)
                def _():
                    issue(c + 2, b)
        return ()

    jax.lax.fori_loop(0, (NCHK + 1) // 2, pair, ())
    plsc.subcore_barrier()

    out0 = sid * ROWS_PT
    pltpu.sync_copy(accr.at[pl.ds(out0, ROWS_PT)],
                    ro_hbm.at[cid, pl.ds(out0, ROWS_PT)])


def _scatter_r_stage(r, idxd3):
    return pl.kernel(
        _scatter_r_body,
        mesh=_sc_mesh(),
        compiler_params=pltpu.CompilerParams(needs_layout_passes=False),
        out_type=jax.ShapeDtypeStruct((SC_NC, NPAD, HID), jnp.float32),
        scratch_types=[pltpu.VMEM((NCHK, CHK), jnp.int32),
                       pltpu.VMEM((CHK, HID), jnp.float32),
                       pltpu.VMEM((CHK, HID), jnp.float32),
                       pltpu.VMEM((SC_L, HID), jnp.float32),
                       pltpu.VMEM_SHARED((NPAD, HID), jnp.float32),
                       pltpu.SemaphoreType.DMA,
                       pltpu.SemaphoreType.DMA,
                       pltpu.SemaphoreType.DMA,
                       pltpu.SemaphoreType.DMA],
    )(r, idxd3)


def _scatter_p_body(cf_hbm, idxd_hbm, idxs_hbm, px_hbm, py_hbm, pz_hbm,
                    px_o, py_o, pz_o, pc_o,
                    idxd_v, idxs_v, bufc, pxv, pyv, pzv,
                    accx, accy, accz, accc, sem):
    cid = jax.lax.axis_index("c")
    sid = jax.lax.axis_index("s")
    wid = cid * SC_NS + sid
    pltpu.sync_copy(idxd_hbm.at[wid], idxd_v)
    pltpu.sync_copy(idxs_hbm.at[wid], idxs_v)
    pltpu.sync_copy(cf_hbm.at[pl.ds(wid * EPW, EPW)], bufc)
    pltpu.sync_copy(px_hbm, pxv)
    pltpu.sync_copy(py_hbm, pyv)
    pltpu.sync_copy(pz_hbm, pzv)

    zeros16 = jnp.zeros((SC_L,), jnp.float32)

    def zacc(k, _):
        sl = pl.ds(k * SC_L, SC_L)
        accx[sl] = zeros16
        accy[sl] = zeros16
        accz[sl] = zeros16
        accc[sl] = zeros16
        return ()

    jax.lax.fori_loop(0, NND // SC_L, zacc, ())

    ones16 = jnp.full((SC_L,), 1.0, jnp.float32)

    def jstep(j, _):
        sl = pl.ds(j * SC_L, SC_L)
        idv = idxd_v[sl]
        isv = idxs_v[sl]
        cf = bufc[sl]
        trx = (plsc.load_gather(pxv, [isv]) - plsc.load_gather(pxv, [idv])) * cf
        try_ = (plsc.load_gather(pyv, [isv]) - plsc.load_gather(pyv, [idv])) * cf
        trz = (plsc.load_gather(pzv, [isv]) - plsc.load_gather(pzv, [idv])) * cf
        plsc.addupdate_scatter(accx, [idv], trx)
        plsc.addupdate_scatter(accy, [idv], try_)
        plsc.addupdate_scatter(accz, [idv], trz)
        plsc.addupdate_scatter(accc, [idv], ones16)
        return ()

    jax.lax.fori_loop(0, EPW // SC_L, jstep, ())

    pltpu.sync_copy(accx, px_o.at[wid])
    pltpu.sync_copy(accy, py_o.at[wid])
    pltpu.sync_copy(accz, pz_o.at[wid])
    pltpu.sync_copy(accc, pc_o.at[wid])


def _scatter_p_stage(coef, idxd2, idxs2, px, py, pz):
    return pl.kernel(
        _scatter_p_body,
        mesh=_sc_mesh(),
        compiler_params=pltpu.CompilerParams(needs_layout_passes=False),
        out_type=[jax.ShapeDtypeStruct((NWRK, NND), jnp.float32),
                  jax.ShapeDtypeStruct((NWRK, NND), jnp.float32),
                  jax.ShapeDtypeStruct((NWRK, NND), jnp.float32),
                  jax.ShapeDtypeStruct((NWRK, NND), jnp.float32)],
        scratch_types=[pltpu.VMEM((EPW,), jnp.int32),
                       pltpu.VMEM((EPW,), jnp.int32),
                       pltpu.VMEM((EPW,), jnp.float32),
                       pltpu.VMEM((NND,), jnp.float32),
                       pltpu.VMEM((NND,), jnp.float32),
                       pltpu.VMEM((NND,), jnp.float32),
                       pltpu.VMEM((NND,), jnp.float32),
                       pltpu.VMEM((NND,), jnp.float32),
                       pltpu.VMEM((NND,), jnp.float32),
                       pltpu.VMEM((NND,), jnp.float32),
                       pltpu.SemaphoreType.DMA],
    )(coef, idxd2, idxs2, px, py, pz)


def _node_body(h_ref, pos_ref, R_ref, px_ref, py_ref, pz_ref, pc_ref,
               We2_ref, be2_ref, Wn1a_ref, Wn1b_ref, bn1_ref,
               Wn2_ref, bn2_ref, Wa_ref, Wb_ref, be1_ref,
               h_out_ref, pos_out_ref, ta_ref, tb_ref):
    R = R_ref[0] + R_ref[1]
    psx = jnp.sum(px_ref[...], axis=1, keepdims=True)
    psy = jnp.sum(py_ref[...], axis=1, keepdims=True)
    psz = jnp.sum(pz_ref[...], axis=1, keepdims=True)
    cnt = jnp.sum(pc_ref[...], axis=1, keepdims=True)
    agg = _hdot(R, _bf(We2_ref[...])) + cnt * be2_ref[...]
    h = h_ref[...]
    mid = jax.nn.relu(_bdot(h, Wn1a_ref[...])
                      + _bdot(agg, Wn1b_ref[...])
                      + bn1_ref[...])
    h_new = h + _bdot(mid, Wn2_ref[...]) + bn2_ref[...]
    h_out_ref[...] = h_new
    lane = jax.lax.broadcasted_iota(jnp.int32, (BLKN, 16), 1)
    row = (jax.lax.broadcasted_iota(jnp.int32, (BLKN, 16), 0)
           + pl.program_id(0) * BLKN)
    ps = (jnp.where(lane == 0, psx, 0.0) + jnp.where(lane == 1, psy, 0.0)
          + jnp.where(lane == 2, psz, 0.0))
    upd = ps / jnp.clip(cnt, 1.0, None)
    pos_out_ref[...] = pos_ref[...] + jnp.where(row >= NPROT, upd, 0.0)
    ta_ref[...] = _bdot(h_new, Wa_ref[...]) + be1_ref[...]
    tb_ref[...] = _bdot(h_new, Wb_ref[...])


def _node_update(h, pos, RO, pxt, pyt, pzt, pct, We2, be2,
                 Wn1a, Wn1b, bn1, Wn2, bn2, Wa, Wb, be1):
    nblk = NND // BLKN
    full = lambda s: pl.BlockSpec(s, lambda i: (0,) * len(s))
    colw = pl.BlockSpec((BLKN, NWRK), lambda i: (i, 0))
    rowb = pl.BlockSpec((BLKN, HID), lambda i: (i, 0))
    return pl.pallas_call(
        _node_body,
        grid=(nblk,),
        in_specs=[
            rowb,
            pl.BlockSpec((BLKN, 16), lambda i: (i, 0)),
            pl.BlockSpec((2, BLKN, HID), lambda i: (0, i, 0)),
            colw, colw, colw, colw,
            full((HID, HID)), full((1, HID)),
            full((HID, HID)), full((HID, HID)), full((1, HID)),
            full((HID, HID)), full((1, HID)),
            full((HID, HID)), full((HID, HID)), full((1, HID)),
        ],
        out_specs=[rowb,
                   pl.BlockSpec((BLKN, 16), lambda i: (i, 0)),
                   rowb, rowb],
        out_shape=[jax.ShapeDtypeStruct((NND, HID), jnp.float32),
                   jax.ShapeDtypeStruct((NND, 16), jnp.float32),
                   jax.ShapeDtypeStruct((NND, HID), jnp.float32),
                   jax.ShapeDtypeStruct((NND, HID), jnp.float32)],
    )(h, pos, RO, pxt, pyt, pzt, pct, We2, be2, Wn1a, Wn1b, bn1, Wn2, bn2,
      Wa, Wb, be1)


def _head_body(h_ref, pos_ref, Wv_ref, bv_ref, out_ref):
    v = _bdot(h_ref[...], Wv_ref[...]) + bv_ref[...]
    lane = jax.lax.broadcasted_iota(jnp.int32, (NLIG, 16), 1)
    pos3 = jnp.where(lane < 3, pos_ref[...], 0.0)
    out_ref[...] = pos3 + v


def _head(h_l, pos_l, Wv_pad, bv_pad):
    return pl.pallas_call(
        _head_body,
        out_shape=jax.ShapeDtypeStruct((NLIG, 16), jnp.float32),
    )(h_l, pos_l, Wv_pad, bv_pad)


def kernel(protein_pos, protein_v, ligand_pos, ligand_v, batch_protein,
           batch_ligand, time_step, edge_index, W_p, b_p, W_l, b_l,
           Wt1, bt1, Wt2, bt2, We1, be1, We2, be2, Wn1, bn1, Wn2, bn2,
           Wc, bc, Wv, bv):
    f32 = jnp.float32
    ppos = jnp.pad(protein_pos, ((0, 0), (0, 13)))
    lpos = jnp.pad(ligand_pos, ((0, 0), (0, 13)))
    pv = jnp.pad(protein_v, ((0, 0), (0, 32 - PFD)))
    Wp_pad = jnp.pad(W_p, ((0, 32 - PFD), (0, 0)))
    lv = jnp.pad(ligand_v, ((0, 0), (0, 16 - LFD)))
    Wl_pad = jnp.pad(W_l, ((0, 16 - LFD), (0, 0)))
    half = HID // 2
    freq = jnp.exp(jnp.arange(half, dtype=f32)
                   * (-np.log(10000.0) / (half - 1)))[None, :]
    bp2 = batch_protein.astype(jnp.int32)[:, None]
    bl2 = batch_ligand.astype(jnp.int32)[:, None]
    t2 = time_step.astype(jnp.int32)[:, None]

    h, pos, ta, tb = _prologue(
        ppos, pv, lpos, lv, bp2, bl2, t2, freq,
        Wp_pad, b_p[None, :], Wl_pad, b_l[None, :],
        Wt1, bt1[None, :], Wt2, bt2[None, :],
        We1[0, :HID], We1[0, HID:2 * HID], be1[0][None, :])

    src_i = edge_index[0].astype(jnp.int32)
    dst_i = edge_index[1].astype(jnp.int32)
    idxs3 = src_i.reshape(NWRK, NCHK, CHK)
    idxd3 = dst_i.reshape(NWRK, NCHK, CHK)
    idxs2 = src_i.reshape(NWRK, EPW)
    idxd2 = dst_i.reshape(NWRK, EPW)

    for i in range(NLAY):
        px, py, pz = pos[:, 0], pos[:, 1], pos[:, 2]
        u = _gather_stage(ta, tb, px, py, pz, _bf(We1[i, 2 * HID]),
                          idxd3, idxs3)
        r, coef = _edge_stage(u, We2[i], be2[i][None, :],
                              Wc[i, :, 0][None, :], bc[i][None, :])
        RO = _scatter_r_stage(r, idxd3)
        pxt, pyt, pzt, pct = _scatter_p_stage(coef[:, 0], idxd2, idxs2,
                                              px, py, pz)
        j = (i + 1) % NLAY
        h, pos, ta, tb = _node_update(
            h, pos, RO[:, :NND], pxt.T, pyt.T, pzt.T, pct.T,
            We2[i], be2[i][None, :], Wn1[i, :HID], Wn1[i, HID:],
            bn1[i][None, :], Wn2[i], bn2[i][None, :],
            We1[j, :HID], We1[j, HID:2 * HID], be1[j][None, :])

    Wv_pad = jnp.pad(Wv, ((0, 0), (3, 0)))
    bv_pad = jnp.pad(bv, ((3, 0),))[None, :]
    return _head(h[NPROT:], pos[NPROT:], Wv_pad, bv_pad)
